# Initial kernel scaffold; baseline (speedup 1.0000x reference)
import jax
import jax.numpy as jnp
from jax.experimental import pallas as pl


def kernel(x, coefficients):
    raise NotImplementedError("write your pallas kernel here")


# speedup vs baseline: 1.9525x; 1.9525x over previous
import jax
import jax.numpy as jnp
from jax.experimental import pallas as pl
from jax.experimental.pallas import tpu as pltpu

_ROWS_PER_BLOCK = 512


def _body(c_ref, x_ref, o_ref):
    x = x_ref[...]
    xc = jnp.clip(x, -1.0, 1.0)
    c0 = c_ref[0]
    c1 = c_ref[1]
    c2 = c_ref[2]
    c3 = c_ref[3]
    c4 = c_ref[4]
    d0 = c1 - c0
    d1 = c2 - c1
    d2 = c3 - c2
    d3 = c4 - c3
    a0 = c0 + 2.0 * d0
    a1 = c1 + d1
    a2 = c2
    a3 = c3 - d3
    lt_m05 = xc < -0.5
    lt_0 = xc < 0.0
    lt_05 = xc < 0.5
    a = jnp.where(lt_m05, a0, jnp.where(lt_0, a1, jnp.where(lt_05, a2, a3)))
    b = jnp.where(
        lt_m05, 2.0 * d0, jnp.where(lt_0, 2.0 * d1, jnp.where(lt_05, 2.0 * d2, 2.0 * d3))
    )
    out = a + b * xc
    o_ref[...] = jnp.where(xc >= 1.0, 0.0, out)


def kernel(x, coefficients):
    orig_shape = x.shape
    x2 = x.reshape(-1, orig_shape[-1])
    rows, cols = x2.shape
    grid = (rows // _ROWS_PER_BLOCK,)
    out = pl.pallas_call(
        _body,
        grid=grid,
        in_specs=[
            pl.BlockSpec(memory_space=pltpu.SMEM),
            pl.BlockSpec((_ROWS_PER_BLOCK, cols), lambda i: (i, 0)),
        ],
        out_specs=pl.BlockSpec((_ROWS_PER_BLOCK, cols), lambda i: (i, 0)),
        out_shape=jax.ShapeDtypeStruct((rows, cols), x.dtype),
    )(coefficients, x2)
    return out.reshape(orig_shape)
